# baseline (speedup 1.0000x reference)
import jax
import jax.numpy as jnp
from jax import lax
from jax.experimental import pallas as pl
from jax.experimental.pallas import tpu as pltpu
from jax.experimental.pallas import tpu_sc as plsc

_EPS = 1e-5

_M = 500000
_B = 16384
_D = 64

_NC = 2
_NS = 16
_HALF = _M // _NC
_REP_ROWS = _HALF + 128
_ACC_ROWS = _B + 128
_TILE_B = _B // _NS
_SCAT_B = _B // (_NC * _NS)
_CH = 128
_NJ = _TILE_B // _CH
_NJ3 = _SCAT_B // _CH
_LANES = 16

_vector_mesh = plsc.VectorSubcoreMesh(core_axis_name="c", subcore_axis_name="s")
_sc_params = pltpu.CompilerParams(use_tc_tiling_on_sc=True)
_ENABLE_GT_GATHER = False
_ENABLE_BARRIER = False
_BISECT_STAGE = 1


def _ln_body(x_ref, w_ref, b_ref, o_ref):
    x = x_ref[...]
    mean = jnp.mean(x, axis=-1, keepdims=True)
    xc = x - mean
    var = jnp.mean(xc * xc, axis=-1, keepdims=True)
    o_ref[...] = xc * lax.rsqrt(var + _EPS) * w_ref[...] + b_ref[...]


def _dense_ln(x, w, b, rows_per_block):
    m, d = x.shape
    r = rows_per_block
    assert m % r == 0
    return pl.pallas_call(
        _ln_body,
        grid=(m // r,),
        in_specs=[
            pl.BlockSpec((r, d), lambda i: (i, 0)),
            pl.BlockSpec((d,), lambda i: (0,)),
            pl.BlockSpec((d,), lambda i: (0,)),
        ],
        out_specs=pl.BlockSpec((r, d), lambda i: (i, 0)),
        out_shape=jax.ShapeDtypeStruct((m, d), x.dtype),
        compiler_params=pltpu.CompilerParams(
            dimension_semantics=("arbitrary",),
        ),
    )(x, w, b)


def _vrows_body(acc_ref, w_ref, b_ref, o_ref):
    x = acc_ref[0] + acc_ref[1]
    mean = jnp.mean(x, axis=-1, keepdims=True)
    xc = x - mean
    var = jnp.mean(xc * xc, axis=-1, keepdims=True)
    n = xc * lax.rsqrt(var + _EPS) * w_ref[...] + b_ref[...]
    o_ref[...] = jnp.concatenate([n, n], axis=-1)


def _vrows(acc, w, b):
    r = _ACC_ROWS // 8
    return pl.pallas_call(
        _vrows_body,
        grid=(8,),
        in_specs=[
            pl.BlockSpec((2, r, _D), lambda i: (0, i, 0)),
            pl.BlockSpec((_D,), lambda i: (0,)),
            pl.BlockSpec((_D,), lambda i: (0,)),
        ],
        out_specs=pl.BlockSpec((r, 2 * _D), lambda i: (i, 0)),
        out_shape=jax.ShapeDtypeStruct((_ACC_ROWS, 2 * _D), jnp.float32),
        compiler_params=pltpu.CompilerParams(
            dimension_semantics=("arbitrary",),
        ),
    )(acc, w, b)


def _fixup_body(gt_hbm, gh_hbm, idx_hbm, zeros_hbm, acc_hbm, w_hbm,
                idx_v, gslots_v, w2d, slots2d, vals2d, aslots2d, wslots2d,
                rows_v, sem, rep_sh, acc_sh):
    c = lax.axis_index("c")
    s = lax.axis_index("s")
    base = c * _HALF
    p0 = s * _TILE_B
    lanes = lax.iota(jnp.int32, _LANES)
    nk = _CH // _LANES

    pltpu.sync_copy(idx_hbm.at[pl.ds(p0, _TILE_B)], idx_v.at[pl.ds(0, _TILE_B)])

    if _BISECT_STAGE >= 1:
        for j in range(_NJ):
            pltpu.sync_copy(zeros_hbm, acc_sh.at[pl.ds((s * _NJ + j) * _CH, _CH)])

    for j in range(_NJ):
        for k in range(nk):
            o = j * _CH + k * _LANES
            kk = pl.ds(k * _LANES, _LANES)
            u = idx_v[pl.ds(o, _LANES)]
            inr = (u >= base) & (u < base + _HALF)
            dump = k * _LANES + lanes
            occ = p0 + o + lanes
            slots2d[j, kk] = jnp.where(inr, u - base, _HALF + dump)
            vals2d[j, kk] = occ

    if _BISECT_STAGE >= 2:
        for j in range(_NJ):
            pltpu.sync_copy(vals2d.at[j], rep_sh.at[slots2d.at[j]])
    if _BISECT_STAGE >= 1 and _ENABLE_BARRIER:
        plsc.subcore_barrier()

    if _BISECT_STAGE >= 2:
        for j in range(_NJ):
            pltpu.sync_copy(rep_sh.at[slots2d.at[j]], w2d.at[j])
        for j in range(_NJ):
            pltpu.sync_copy(w2d.at[j], w_hbm.at[pl.ds(c * _B + p0 + j * _CH, _CH)])
    for j in range(_NJ):
        for k in range(nk):
            o = j * _CH + k * _LANES
            kk = pl.ds(k * _LANES, _LANES)
            u = idx_v[pl.ds(o, _LANES)]
            inr = (u >= base) & (u < base + _HALF)
            w = w2d[j, kk]
            occ = p0 + o + lanes
            isw = inr & (w == occ)
            dump = k * _LANES + lanes
            aslots2d[j, kk] = jnp.where(inr, w, _B + dump)
            gslots_v[pl.ds(o, _LANES)] = jnp.where(isw, u, occ)
            wslots2d[j, kk] = jnp.where(isw, w, _B + dump)

    if _BISECT_STAGE >= 3:
        for j in range(_NJ):
            pltpu.sync_copy(gh_hbm.at[pl.ds(p0 + j * _CH, _CH)], rows_v)
            pltpu.sync_copy(rows_v, acc_sh.at[aslots2d.at[j]], add=True)

    if _BISECT_STAGE >= 4 and _ENABLE_GT_GATHER:
        for j in range(_NJ):
            @pl.loop(0, _CH)
            def _(r):
                row = gslots_v[pl.ds(j * _CH + r, _LANES)][0]
                pltpu.async_copy(gt_hbm.at[pl.ds(row, 1)],
                                 rows_v.at[pl.ds(r, 1)], sem)

            @pl.loop(0, _CH)
            def _(r):
                pltpu.make_async_copy(gt_hbm.at[pl.ds(0, 1)],
                                      rows_v.at[pl.ds(r, 1)], sem).wait()
            pltpu.sync_copy(rows_v, acc_sh.at[wslots2d.at[j]], add=True)
    if _ENABLE_BARRIER:
        plsc.subcore_barrier()

    if _BISECT_STAGE >= 1:
        for j in range(_NJ):
            sl = pl.ds((s * _NJ + j) * _CH, _CH)
            dsl = pl.ds(c * _ACC_ROWS + (s * _NJ + j) * _CH, _CH)
            pltpu.sync_copy(acc_sh.at[sl], acc_hbm.at[dsl])


def _sc_fixup(G_t, G_h, idx, zeros):
    f = pl.kernel(
        _fixup_body,
        out_type=(
            jax.ShapeDtypeStruct((_NC * _ACC_ROWS, _D), jnp.float32),
            jax.ShapeDtypeStruct((_NC * _B,), jnp.int32),
        ),
        mesh=_vector_mesh,
        scratch_types=[
            pltpu.VMEM((_TILE_B + _LANES,), jnp.int32),
            pltpu.VMEM((_TILE_B + _LANES,), jnp.int32),
            pltpu.VMEM((_NJ, _CH), jnp.int32),
            pltpu.VMEM((_NJ, _CH), jnp.int32),
            pltpu.VMEM((_NJ, _CH), jnp.int32),
            pltpu.VMEM((_NJ, _CH), jnp.int32),
            pltpu.VMEM((_NJ, _CH), jnp.int32),
            pltpu.VMEM((_CH, _D), jnp.float32),
            pltpu.SemaphoreType.DMA,
            pltpu.VMEM_SHARED((_REP_ROWS,), jnp.int32),
            pltpu.VMEM_SHARED((_ACC_ROWS, _D), jnp.float32),
        ],
        compiler_params=_sc_params,
    )
    return f(G_t, G_h, idx, zeros)


def _scatter_body(v2_hbm, idx_hbm, w_hbm, out_ref,
                  idx_v, wsel2d, w1_2d, pair_v, row_v, sem):
    c = lax.axis_index("c")
    s = lax.axis_index("s")
    wid = s * _NC + c
    p0 = wid * _SCAT_B
    nk = _CH // _LANES

    pltpu.sync_copy(idx_hbm.at[pl.ds(p0, _SCAT_B)], idx_v.at[pl.ds(0, _SCAT_B)])
    for j in range(_NJ3):
        pltpu.sync_copy(w_hbm.at[pl.ds(p0 + j * _CH, _CH)], wsel2d.at[j])
        pltpu.sync_copy(w_hbm.at[pl.ds(_B + p0 + j * _CH, _CH)], w1_2d.at[j])

    for j in range(_NJ3):
        for k in range(nk):
            o = j * _CH + k * _LANES
            kk = pl.ds(k * _LANES, _LANES)
            u = idx_v[pl.ds(o, _LANES)]
            w0 = wsel2d[j, kk]
            w1 = w1_2d[j, kk]
            wsel2d[j, kk] = jnp.where(u < _HALF, w0, w1)

    for j in range(_NJ3):
        pltpu.sync_copy(v2_hbm.at[wsel2d.at[j]], pair_v)

        @pl.loop(0, _CH)
        def _(r):
            for q in range(_D // _LANES):
                row_v[r, pl.ds(q * _LANES, _LANES)] = \
                    pair_v[r, pl.ds(q * _LANES, _LANES)]

        @pl.loop(0, _CH)
        def _(r):
            row = idx_v[pl.ds(j * _CH + r, _LANES)][0]
            pltpu.async_copy(row_v.at[pl.ds(r, 1)],
                             out_ref.at[pl.ds(row, 1)], sem)

        @pl.loop(0, _CH)
        def _(r):
            pltpu.make_async_copy(row_v.at[pl.ds(r, 1)],
                                  out_ref.at[pl.ds(r, 1)], sem).wait()


def _sc_scatter(out, v2, idx, w_planes):
    out_ref = jax.new_ref(out)
    f = pl.kernel(
        _scatter_body,
        out_type=(),
        mesh=_vector_mesh,
        scratch_types=[
            pltpu.VMEM((_SCAT_B + _LANES,), jnp.int32),
            pltpu.VMEM((_NJ3, _CH), jnp.int32),
            pltpu.VMEM((_NJ3, _CH), jnp.int32),
            pltpu.VMEM((_CH, 2 * _D), jnp.float32),
            pltpu.VMEM((_CH, _D), jnp.float32),
            pltpu.SemaphoreType.DMA,
        ],
        compiler_params=_sc_params,
    )
    f(v2, idx, w_planes, out_ref)
    return out_ref[...]


def kernel(G_h, G_t, q, h_map_t, ln_weight, ln_bias):
    G_t_new = G_t.at[h_map_t].add(G_h)
    return _dense_ln(G_t_new, ln_weight, ln_bias, 2000)


# speedup vs baseline: 1.1979x; 1.1979x over previous
import jax
import jax.numpy as jnp
from jax import lax
from jax.experimental import pallas as pl
from jax.experimental.pallas import tpu as pltpu
from jax.experimental.pallas import tpu_sc as plsc

_EPS = 1e-5

_M = 500000
_B = 16384
_D = 64

_NC = 2
_NS = 16
_HALF = _M // _NC
_REP_ROWS = _HALF + 128
_ACC_ROWS = _B + 128
_TILE_B = _B // _NS
_SCAT_B = _B // (_NC * _NS)
_CH = 128
_NJ = _TILE_B // _CH
_NJ3 = _SCAT_B // _CH
_LANES = 16

_vector_mesh = plsc.VectorSubcoreMesh(core_axis_name="c", subcore_axis_name="s")
_sc_params = pltpu.CompilerParams(use_tc_tiling_on_sc=True)
_ENABLE_GT_GATHER = False
_ENABLE_BARRIER = False
_BISECT_STAGE = 1


def _ln_body(x_ref, w_ref, b_ref, o_ref):
    x = x_ref[...]
    mean = jnp.mean(x, axis=-1, keepdims=True)
    xc = x - mean
    var = jnp.mean(xc * xc, axis=-1, keepdims=True)
    o_ref[...] = xc * lax.rsqrt(var + _EPS) * w_ref[...] + b_ref[...]


def _dense_ln(x, w, b, rows_per_block):
    m, d = x.shape
    r = rows_per_block
    assert m % r == 0
    return pl.pallas_call(
        _ln_body,
        grid=(m // r,),
        in_specs=[
            pl.BlockSpec((r, d), lambda i: (i, 0)),
            pl.BlockSpec((d,), lambda i: (0,)),
            pl.BlockSpec((d,), lambda i: (0,)),
        ],
        out_specs=pl.BlockSpec((r, d), lambda i: (i, 0)),
        out_shape=jax.ShapeDtypeStruct((m, d), x.dtype),
        compiler_params=pltpu.CompilerParams(
            dimension_semantics=("arbitrary",),
        ),
    )(x, w, b)


def _vrows_body(acc_ref, w_ref, b_ref, o_ref):
    x = acc_ref[0] + acc_ref[1]
    mean = jnp.mean(x, axis=-1, keepdims=True)
    xc = x - mean
    var = jnp.mean(xc * xc, axis=-1, keepdims=True)
    n = xc * lax.rsqrt(var + _EPS) * w_ref[...] + b_ref[...]
    o_ref[...] = jnp.concatenate([n, n], axis=-1)


def _vrows(acc, w, b):
    r = _ACC_ROWS // 8
    return pl.pallas_call(
        _vrows_body,
        grid=(8,),
        in_specs=[
            pl.BlockSpec((2, r, _D), lambda i: (0, i, 0)),
            pl.BlockSpec((_D,), lambda i: (0,)),
            pl.BlockSpec((_D,), lambda i: (0,)),
        ],
        out_specs=pl.BlockSpec((r, 2 * _D), lambda i: (i, 0)),
        out_shape=jax.ShapeDtypeStruct((_ACC_ROWS, 2 * _D), jnp.float32),
        compiler_params=pltpu.CompilerParams(
            dimension_semantics=("arbitrary",),
        ),
    )(acc, w, b)


def _fixup_body(gt_hbm, gh_hbm, idx_hbm, zeros_hbm, acc_hbm, w_hbm,
                idx_v, gslots_v, w2d, slots2d, vals2d, aslots2d, wslots2d,
                rows_v, sem, rep_sh, acc_sh):
    c = lax.axis_index("c")
    s = lax.axis_index("s")
    base = c * _HALF
    p0 = s * _TILE_B
    lanes = lax.iota(jnp.int32, _LANES)
    nk = _CH // _LANES

    pltpu.sync_copy(idx_hbm.at[pl.ds(p0, _TILE_B)], idx_v.at[pl.ds(0, _TILE_B)])

    if _BISECT_STAGE >= 1:
        for j in range(_NJ):
            pltpu.sync_copy(zeros_hbm, acc_sh.at[pl.ds((s * _NJ + j) * _CH, _CH)])

    for j in range(_NJ):
        for k in range(nk):
            o = j * _CH + k * _LANES
            kk = pl.ds(k * _LANES, _LANES)
            u = idx_v[pl.ds(o, _LANES)]
            inr = (u >= base) & (u < base + _HALF)
            dump = k * _LANES + lanes
            occ = p0 + o + lanes
            slots2d[j, kk] = jnp.where(inr, u - base, _HALF + dump)
            vals2d[j, kk] = occ

    if _BISECT_STAGE >= 2:
        for j in range(_NJ):
            pltpu.sync_copy(vals2d.at[j], rep_sh.at[slots2d.at[j]])
    if _BISECT_STAGE >= 1 and _ENABLE_BARRIER:
        plsc.subcore_barrier()

    if _BISECT_STAGE >= 2:
        for j in range(_NJ):
            pltpu.sync_copy(rep_sh.at[slots2d.at[j]], w2d.at[j])
        for j in range(_NJ):
            pltpu.sync_copy(w2d.at[j], w_hbm.at[pl.ds(c * _B + p0 + j * _CH, _CH)])
    for j in range(_NJ):
        for k in range(nk):
            o = j * _CH + k * _LANES
            kk = pl.ds(k * _LANES, _LANES)
            u = idx_v[pl.ds(o, _LANES)]
            inr = (u >= base) & (u < base + _HALF)
            w = w2d[j, kk]
            occ = p0 + o + lanes
            isw = inr & (w == occ)
            dump = k * _LANES + lanes
            aslots2d[j, kk] = jnp.where(inr, w, _B + dump)
            gslots_v[pl.ds(o, _LANES)] = jnp.where(isw, u, occ)
            wslots2d[j, kk] = jnp.where(isw, w, _B + dump)

    if _BISECT_STAGE >= 3:
        for j in range(_NJ):
            pltpu.sync_copy(gh_hbm.at[pl.ds(p0 + j * _CH, _CH)], rows_v)
            pltpu.sync_copy(rows_v, acc_sh.at[aslots2d.at[j]], add=True)

    if _BISECT_STAGE >= 4 and _ENABLE_GT_GATHER:
        for j in range(_NJ):
            @pl.loop(0, _CH)
            def _(r):
                row = gslots_v[pl.ds(j * _CH + r, _LANES)][0]
                pltpu.async_copy(gt_hbm.at[pl.ds(row, 1)],
                                 rows_v.at[pl.ds(r, 1)], sem)

            @pl.loop(0, _CH)
            def _(r):
                pltpu.make_async_copy(gt_hbm.at[pl.ds(0, 1)],
                                      rows_v.at[pl.ds(r, 1)], sem).wait()
            pltpu.sync_copy(rows_v, acc_sh.at[wslots2d.at[j]], add=True)
    if _ENABLE_BARRIER:
        plsc.subcore_barrier()

    if _BISECT_STAGE >= 1:
        for j in range(_NJ):
            sl = pl.ds((s * _NJ + j) * _CH, _CH)
            dsl = pl.ds(c * _ACC_ROWS + (s * _NJ + j) * _CH, _CH)
            pltpu.sync_copy(acc_sh.at[sl], acc_hbm.at[dsl])


def _sc_fixup(G_t, G_h, idx, zeros):
    f = pl.kernel(
        _fixup_body,
        out_type=(
            jax.ShapeDtypeStruct((_NC * _ACC_ROWS, _D), jnp.float32),
            jax.ShapeDtypeStruct((_NC * _B,), jnp.int32),
        ),
        mesh=_vector_mesh,
        scratch_types=[
            pltpu.VMEM((_TILE_B + _LANES,), jnp.int32),
            pltpu.VMEM((_TILE_B + _LANES,), jnp.int32),
            pltpu.VMEM((_NJ, _CH), jnp.int32),
            pltpu.VMEM((_NJ, _CH), jnp.int32),
            pltpu.VMEM((_NJ, _CH), jnp.int32),
            pltpu.VMEM((_NJ, _CH), jnp.int32),
            pltpu.VMEM((_NJ, _CH), jnp.int32),
            pltpu.VMEM((_CH, _D), jnp.float32),
            pltpu.SemaphoreType.DMA,
            pltpu.VMEM_SHARED((_REP_ROWS,), jnp.int32),
            pltpu.VMEM_SHARED((_ACC_ROWS, _D), jnp.float32),
        ],
        compiler_params=_sc_params,
    )
    return f(G_t, G_h, idx, zeros)


def _scatter_body(v2_hbm, idx_hbm, w_hbm, out_ref,
                  idx_v, wsel2d, w1_2d, pair_v, row_v, sem):
    c = lax.axis_index("c")
    s = lax.axis_index("s")
    wid = s * _NC + c
    p0 = wid * _SCAT_B
    nk = _CH // _LANES

    pltpu.sync_copy(idx_hbm.at[pl.ds(p0, _SCAT_B)], idx_v.at[pl.ds(0, _SCAT_B)])
    for j in range(_NJ3):
        pltpu.sync_copy(w_hbm.at[pl.ds(p0 + j * _CH, _CH)], wsel2d.at[j])
        pltpu.sync_copy(w_hbm.at[pl.ds(_B + p0 + j * _CH, _CH)], w1_2d.at[j])

    for j in range(_NJ3):
        for k in range(nk):
            o = j * _CH + k * _LANES
            kk = pl.ds(k * _LANES, _LANES)
            u = idx_v[pl.ds(o, _LANES)]
            w0 = wsel2d[j, kk]
            w1 = w1_2d[j, kk]
            wsel2d[j, kk] = jnp.where(u < _HALF, w0, w1)

    for j in range(_NJ3):
        pltpu.sync_copy(v2_hbm.at[wsel2d.at[j]], pair_v)

        @pl.loop(0, _CH)
        def _(r):
            for q in range(_D // _LANES):
                row_v[r, pl.ds(q * _LANES, _LANES)] = \
                    pair_v[r, pl.ds(q * _LANES, _LANES)]

        @pl.loop(0, _CH)
        def _(r):
            row = idx_v[pl.ds(j * _CH + r, _LANES)][0]
            pltpu.async_copy(row_v.at[pl.ds(r, 1)],
                             out_ref.at[pl.ds(row, 1)], sem)

        @pl.loop(0, _CH)
        def _(r):
            pltpu.make_async_copy(row_v.at[pl.ds(r, 1)],
                                  out_ref.at[pl.ds(r, 1)], sem).wait()


def _sc_scatter(out, v2, idx, w_planes):
    out_ref = jax.new_ref(out)
    f = pl.kernel(
        _scatter_body,
        out_type=(),
        mesh=_vector_mesh,
        scratch_types=[
            pltpu.VMEM((_SCAT_B + _LANES,), jnp.int32),
            pltpu.VMEM((_NJ3, _CH), jnp.int32),
            pltpu.VMEM((_NJ3, _CH), jnp.int32),
            pltpu.VMEM((_CH, 2 * _D), jnp.float32),
            pltpu.VMEM((_CH, _D), jnp.float32),
            pltpu.SemaphoreType.DMA,
        ],
        compiler_params=_sc_params,
    )
    f(v2, idx, w_planes, out_ref)
    return out_ref[...]


def kernel(G_h, G_t, q, h_map_t, ln_weight, ln_bias):
    G_t_new = G_t.at[h_map_t].add(G_h)
    return _dense_ln(G_t_new, ln_weight, ln_bias, 10000)


# speedup vs baseline: 1.2263x; 1.0237x over previous
import jax
import jax.numpy as jnp
from jax import lax
from jax.experimental import pallas as pl
from jax.experimental.pallas import tpu as pltpu
from jax.experimental.pallas import tpu_sc as plsc

_EPS = 1e-5

_M = 500000
_B = 16384
_D = 64

_NC = 2
_NS = 16
_HALF = _M // _NC
_REP_ROWS = _HALF + 128
_ACC_ROWS = _B + 128
_TILE_B = _B // _NS
_SCAT_B = _B // (_NC * _NS)
_CH = 128
_NJ = _TILE_B // _CH
_NJ3 = _SCAT_B // _CH
_LANES = 16

_vector_mesh = plsc.VectorSubcoreMesh(core_axis_name="c", subcore_axis_name="s")
_sc_params = pltpu.CompilerParams(use_tc_tiling_on_sc=True)
_ENABLE_GT_GATHER = False
_ENABLE_BARRIER = False
_BISECT_STAGE = 1


def _ln_body(x_ref, w_ref, b_ref, o_ref):
    x = x_ref[...]
    mean = jnp.mean(x, axis=-1, keepdims=True)
    xc = x - mean
    var = jnp.mean(xc * xc, axis=-1, keepdims=True)
    o_ref[...] = xc * lax.rsqrt(var + _EPS) * w_ref[...] + b_ref[...]


def _dense_ln(x, w, b, rows_per_block):
    m, d = x.shape
    r = rows_per_block
    assert m % r == 0
    return pl.pallas_call(
        _ln_body,
        grid=(m // r,),
        in_specs=[
            pl.BlockSpec((r, d), lambda i: (i, 0)),
            pl.BlockSpec((d,), lambda i: (0,)),
            pl.BlockSpec((d,), lambda i: (0,)),
        ],
        out_specs=pl.BlockSpec((r, d), lambda i: (i, 0)),
        out_shape=jax.ShapeDtypeStruct((m, d), x.dtype),
        compiler_params=pltpu.CompilerParams(
            dimension_semantics=("arbitrary",),
        ),
    )(x, w, b)


def _vrows_body(acc_ref, w_ref, b_ref, o_ref):
    x = acc_ref[0] + acc_ref[1]
    mean = jnp.mean(x, axis=-1, keepdims=True)
    xc = x - mean
    var = jnp.mean(xc * xc, axis=-1, keepdims=True)
    n = xc * lax.rsqrt(var + _EPS) * w_ref[...] + b_ref[...]
    o_ref[...] = jnp.concatenate([n, n], axis=-1)


def _vrows(acc, w, b):
    r = _ACC_ROWS // 8
    return pl.pallas_call(
        _vrows_body,
        grid=(8,),
        in_specs=[
            pl.BlockSpec((2, r, _D), lambda i: (0, i, 0)),
            pl.BlockSpec((_D,), lambda i: (0,)),
            pl.BlockSpec((_D,), lambda i: (0,)),
        ],
        out_specs=pl.BlockSpec((r, 2 * _D), lambda i: (i, 0)),
        out_shape=jax.ShapeDtypeStruct((_ACC_ROWS, 2 * _D), jnp.float32),
        compiler_params=pltpu.CompilerParams(
            dimension_semantics=("arbitrary",),
        ),
    )(acc, w, b)


def _fixup_body(gt_hbm, gh_hbm, idx_hbm, zeros_hbm, acc_hbm, w_hbm,
                idx_v, gslots_v, w2d, slots2d, vals2d, aslots2d, wslots2d,
                rows_v, sem, rep_sh, acc_sh):
    c = lax.axis_index("c")
    s = lax.axis_index("s")
    base = c * _HALF
    p0 = s * _TILE_B
    lanes = lax.iota(jnp.int32, _LANES)
    nk = _CH // _LANES

    pltpu.sync_copy(idx_hbm.at[pl.ds(p0, _TILE_B)], idx_v.at[pl.ds(0, _TILE_B)])

    if _BISECT_STAGE >= 1:
        for j in range(_NJ):
            pltpu.sync_copy(zeros_hbm, acc_sh.at[pl.ds((s * _NJ + j) * _CH, _CH)])

    for j in range(_NJ):
        for k in range(nk):
            o = j * _CH + k * _LANES
            kk = pl.ds(k * _LANES, _LANES)
            u = idx_v[pl.ds(o, _LANES)]
            inr = (u >= base) & (u < base + _HALF)
            dump = k * _LANES + lanes
            occ = p0 + o + lanes
            slots2d[j, kk] = jnp.where(inr, u - base, _HALF + dump)
            vals2d[j, kk] = occ

    if _BISECT_STAGE >= 2:
        for j in range(_NJ):
            pltpu.sync_copy(vals2d.at[j], rep_sh.at[slots2d.at[j]])
    if _BISECT_STAGE >= 1 and _ENABLE_BARRIER:
        plsc.subcore_barrier()

    if _BISECT_STAGE >= 2:
        for j in range(_NJ):
            pltpu.sync_copy(rep_sh.at[slots2d.at[j]], w2d.at[j])
        for j in range(_NJ):
            pltpu.sync_copy(w2d.at[j], w_hbm.at[pl.ds(c * _B + p0 + j * _CH, _CH)])
    for j in range(_NJ):
        for k in range(nk):
            o = j * _CH + k * _LANES
            kk = pl.ds(k * _LANES, _LANES)
            u = idx_v[pl.ds(o, _LANES)]
            inr = (u >= base) & (u < base + _HALF)
            w = w2d[j, kk]
            occ = p0 + o + lanes
            isw = inr & (w == occ)
            dump = k * _LANES + lanes
            aslots2d[j, kk] = jnp.where(inr, w, _B + dump)
            gslots_v[pl.ds(o, _LANES)] = jnp.where(isw, u, occ)
            wslots2d[j, kk] = jnp.where(isw, w, _B + dump)

    if _BISECT_STAGE >= 3:
        for j in range(_NJ):
            pltpu.sync_copy(gh_hbm.at[pl.ds(p0 + j * _CH, _CH)], rows_v)
            pltpu.sync_copy(rows_v, acc_sh.at[aslots2d.at[j]], add=True)

    if _BISECT_STAGE >= 4 and _ENABLE_GT_GATHER:
        for j in range(_NJ):
            @pl.loop(0, _CH)
            def _(r):
                row = gslots_v[pl.ds(j * _CH + r, _LANES)][0]
                pltpu.async_copy(gt_hbm.at[pl.ds(row, 1)],
                                 rows_v.at[pl.ds(r, 1)], sem)

            @pl.loop(0, _CH)
            def _(r):
                pltpu.make_async_copy(gt_hbm.at[pl.ds(0, 1)],
                                      rows_v.at[pl.ds(r, 1)], sem).wait()
            pltpu.sync_copy(rows_v, acc_sh.at[wslots2d.at[j]], add=True)
    if _ENABLE_BARRIER:
        plsc.subcore_barrier()

    if _BISECT_STAGE >= 1:
        for j in range(_NJ):
            sl = pl.ds((s * _NJ + j) * _CH, _CH)
            dsl = pl.ds(c * _ACC_ROWS + (s * _NJ + j) * _CH, _CH)
            pltpu.sync_copy(acc_sh.at[sl], acc_hbm.at[dsl])


def _sc_fixup(G_t, G_h, idx, zeros):
    f = pl.kernel(
        _fixup_body,
        out_type=(
            jax.ShapeDtypeStruct((_NC * _ACC_ROWS, _D), jnp.float32),
            jax.ShapeDtypeStruct((_NC * _B,), jnp.int32),
        ),
        mesh=_vector_mesh,
        scratch_types=[
            pltpu.VMEM((_TILE_B + _LANES,), jnp.int32),
            pltpu.VMEM((_TILE_B + _LANES,), jnp.int32),
            pltpu.VMEM((_NJ, _CH), jnp.int32),
            pltpu.VMEM((_NJ, _CH), jnp.int32),
            pltpu.VMEM((_NJ, _CH), jnp.int32),
            pltpu.VMEM((_NJ, _CH), jnp.int32),
            pltpu.VMEM((_NJ, _CH), jnp.int32),
            pltpu.VMEM((_CH, _D), jnp.float32),
            pltpu.SemaphoreType.DMA,
            pltpu.VMEM_SHARED((_REP_ROWS,), jnp.int32),
            pltpu.VMEM_SHARED((_ACC_ROWS, _D), jnp.float32),
        ],
        compiler_params=_sc_params,
    )
    return f(G_t, G_h, idx, zeros)


def _scatter_body(v2_hbm, idx_hbm, w_hbm, out_ref,
                  idx_v, wsel2d, w1_2d, pair_v, row_v, sem):
    c = lax.axis_index("c")
    s = lax.axis_index("s")
    wid = s * _NC + c
    p0 = wid * _SCAT_B
    nk = _CH // _LANES

    pltpu.sync_copy(idx_hbm.at[pl.ds(p0, _SCAT_B)], idx_v.at[pl.ds(0, _SCAT_B)])
    for j in range(_NJ3):
        pltpu.sync_copy(w_hbm.at[pl.ds(p0 + j * _CH, _CH)], wsel2d.at[j])
        pltpu.sync_copy(w_hbm.at[pl.ds(_B + p0 + j * _CH, _CH)], w1_2d.at[j])

    for j in range(_NJ3):
        for k in range(nk):
            o = j * _CH + k * _LANES
            kk = pl.ds(k * _LANES, _LANES)
            u = idx_v[pl.ds(o, _LANES)]
            w0 = wsel2d[j, kk]
            w1 = w1_2d[j, kk]
            wsel2d[j, kk] = jnp.where(u < _HALF, w0, w1)

    for j in range(_NJ3):
        pltpu.sync_copy(v2_hbm.at[wsel2d.at[j]], pair_v)

        @pl.loop(0, _CH)
        def _(r):
            for q in range(_D // _LANES):
                row_v[r, pl.ds(q * _LANES, _LANES)] = \
                    pair_v[r, pl.ds(q * _LANES, _LANES)]

        @pl.loop(0, _CH)
        def _(r):
            row = idx_v[pl.ds(j * _CH + r, _LANES)][0]
            pltpu.async_copy(row_v.at[pl.ds(r, 1)],
                             out_ref.at[pl.ds(row, 1)], sem)

        @pl.loop(0, _CH)
        def _(r):
            pltpu.make_async_copy(row_v.at[pl.ds(r, 1)],
                                  out_ref.at[pl.ds(r, 1)], sem).wait()


def _sc_scatter(out, v2, idx, w_planes):
    out_ref = jax.new_ref(out)
    f = pl.kernel(
        _scatter_body,
        out_type=(),
        mesh=_vector_mesh,
        scratch_types=[
            pltpu.VMEM((_SCAT_B + _LANES,), jnp.int32),
            pltpu.VMEM((_NJ3, _CH), jnp.int32),
            pltpu.VMEM((_NJ3, _CH), jnp.int32),
            pltpu.VMEM((_CH, 2 * _D), jnp.float32),
            pltpu.VMEM((_CH, _D), jnp.float32),
            pltpu.SemaphoreType.DMA,
        ],
        compiler_params=_sc_params,
    )
    f(v2, idx, w_planes, out_ref)
    return out_ref[...]


def kernel(G_h, G_t, q, h_map_t, ln_weight, ln_bias):
    G_t_new = G_t.at[h_map_t].add(G_h)
    return _dense_ln(G_t_new, ln_weight, ln_bias, 20000)
